# baseline (device time: 185605 ns/iter reference)
import jax
import jax.numpy as jnp
from jax import lax
from jax.experimental import pallas as pl
from jax.experimental.pallas import tpu as pltpu

N_DEV = 4


def kernel(x, w_mat, scale_x, scale_w):
    m, k_shard = x.shape
    _, n = w_mat.shape
    kh = k_shard // 2
    mc = m // N_DEV

    x8a = x[:, :kh].astype(jnp.float8_e4m3fn)
    x8b = x[:, kh:].astype(jnp.float8_e4m3fn)
    w8 = w_mat.astype(jnp.float8_e4m3fn)

    def body(xa_ref, xb_ref, w_ref, sx_ref, sw_ref, out_ref,
             xf, wf, acc, tmp,
             xs_r, ws_r, xs_l, ws_l,
             xr_r, wr_r, xr_l, wr_l,
             dma_sem):
        my = lax.axis_index("i")
        left = (my + N_DEV - 1) % N_DEV
        right = (my + 1) % N_DEV

        def c(k):
            return (my + 8 * N_DEV + k) % N_DEV

        barrier_sem = pltpu.get_barrier_semaphore()
        for nbr in [left, right]:
            pl.semaphore_signal(
                barrier_sem, inc=1,
                device_id=(nbr,), device_id_type=pl.DeviceIdType.MESH,
            )
        pl.semaphore_wait(barrier_sem, 2)

        pending = []

        def send(src, dst, ssem, rsem, to):
            rdma = pltpu.make_async_remote_copy(
                src_ref=src, dst_ref=dst, send_sem=ssem, recv_sem=rsem,
                device_id=(to,), device_id_type=pl.DeviceIdType.MESH,
            )
            rdma.start()
            pending.append(rdma)
            return rdma

        def accum2(jr, jl, init):
            for r in range(N_DEV):
                d = jnp.dot(
                    xf[jr, 0, pl.ds(r * mc, mc), :], wf[jr, 0],
                    preferred_element_type=jnp.float32,
                ) + jnp.dot(
                    xf[jl, 1, pl.ds(r * mc, mc), :], wf[jl, 1],
                    preferred_element_type=jnp.float32,
                )
                if init:
                    acc[pl.ds(r * mc, mc), :] = d.astype(jnp.bfloat16)
                else:
                    acc[pl.ds(r * mc, mc), :] = (
                        acc[pl.ds(r * mc, mc), :].astype(jnp.float32) + d
                    ).astype(jnp.bfloat16)

        recvs = []
        for s in range(N_DEV - 1):
            if s == 0:
                sr_x, sr_w = xa_ref.at[:, :], w_ref.at[pl.ds(0, kh), :]
                sl_x, sl_w = xb_ref.at[:, :], w_ref.at[pl.ds(kh, kh), :]
            else:
                sr_x, sr_w = xf.at[c(-s), 0], wf.at[c(-s), 0]
                sl_x, sl_w = xf.at[c(s), 1], wf.at[c(s), 1]
            step = [
                send(sr_x, xf.at[c(-s), 0], xs_r.at[s], xr_r.at[s], right),
                send(sr_w, wf.at[c(-s), 0], ws_r.at[s], wr_r.at[s], right),
                send(sl_x, xf.at[c(s), 1], xs_l.at[s], xr_l.at[s], left),
                send(sl_w, wf.at[c(s), 1], ws_l.at[s], wr_l.at[s], left),
            ]
            if s == 0:
                for r in range(N_DEV):
                    acc[pl.ds(r * mc, mc), :] = (
                        jnp.dot(
                            xa_ref[pl.ds(r * mc, mc), :],
                            w_ref[pl.ds(0, kh), :],
                            preferred_element_type=jnp.float32,
                        )
                        + jnp.dot(
                            xb_ref[pl.ds(r * mc, mc), :],
                            w_ref[pl.ds(kh, kh), :],
                            preferred_element_type=jnp.float32,
                        )
                    ).astype(jnp.bfloat16)
            else:
                accum2(c(-s), c(s), False)
            for rdma in step:
                rdma.wait_recv()
            recvs.append(step)

        scale = sx_ref[0] * sw_ref[0]
        prev_cp = None
        for r in range(N_DEV):
            t = (
                acc[pl.ds(r * mc, mc), :].astype(jnp.float32)
                + jnp.dot(
                    xf[c(1), 0, pl.ds(r * mc, mc), :], wf[c(1), 0],
                    preferred_element_type=jnp.float32,
                )
                + jnp.dot(
                    xf[c(-1), 1, pl.ds(r * mc, mc), :], wf[c(-1), 1],
                    preferred_element_type=jnp.float32,
                )
            )
            y = t * scale
            z = y * (1.0 / (1.0 + jnp.exp(-y)))
            if prev_cp is not None:
                prev_cp.wait()
            tmp[...] = z
            prev_cp = pltpu.make_async_copy(
                tmp, out_ref.at[pl.ds(r * mc, mc), :], dma_sem
            )
            prev_cp.start()
        prev_cp.wait()

        for rdma in pending:
            rdma.wait_send()

    out_shape = jax.ShapeDtypeStruct((m, n), jnp.float32)
    return pl.pallas_call(
        body,
        out_shape=out_shape,
        in_specs=[
            pl.BlockSpec(memory_space=pltpu.VMEM),
            pl.BlockSpec(memory_space=pltpu.VMEM),
            pl.BlockSpec(memory_space=pltpu.VMEM),
            pl.BlockSpec(memory_space=pltpu.SMEM),
            pl.BlockSpec(memory_space=pltpu.SMEM),
        ],
        out_specs=pl.BlockSpec(memory_space=pltpu.MemorySpace.HBM),
        scratch_shapes=[
            pltpu.VMEM((N_DEV, 2, m, kh), jnp.float8_e4m3fn),
            pltpu.VMEM((N_DEV, 2, kh, n), jnp.float8_e4m3fn),
            pltpu.VMEM((m, n), jnp.bfloat16),
            pltpu.VMEM((mc, n), jnp.float32),
            pltpu.SemaphoreType.DMA((N_DEV - 1,)),
            pltpu.SemaphoreType.DMA((N_DEV - 1,)),
            pltpu.SemaphoreType.DMA((N_DEV - 1,)),
            pltpu.SemaphoreType.DMA((N_DEV - 1,)),
            pltpu.SemaphoreType.DMA((N_DEV - 1,)),
            pltpu.SemaphoreType.DMA((N_DEV - 1,)),
            pltpu.SemaphoreType.DMA((N_DEV - 1,)),
            pltpu.SemaphoreType.DMA((N_DEV - 1,)),
            pltpu.SemaphoreType.DMA,
        ],
        compiler_params=pltpu.CompilerParams(
            collective_id=0,
            vmem_limit_bytes=62 * 1024 * 1024,
        ),
    )(x8a, x8b, w8, scale_x, scale_w)


# device time: 171496 ns/iter; 1.0823x vs baseline; 1.0823x over previous
import jax
import jax.numpy as jnp
from jax import lax
from jax.experimental import pallas as pl
from jax.experimental.pallas import tpu as pltpu

N_DEV = 4


def kernel(x, w_mat, scale_x, scale_w):
    m, k_shard = x.shape
    _, n = w_mat.shape
    kh = k_shard // 2
    mc = m // N_DEV

    x8a = x[:, :kh].astype(jnp.float8_e4m3fn)
    x8b = x[:, kh:].astype(jnp.float8_e4m3fn)
    w8 = w_mat.astype(jnp.float8_e4m3fn)

    mh = m // 2

    def body(xa_ref, xb_ref, w_ref, sx_ref, sw_ref, out_ref,
             xf, wf, acc, tmp,
             xsa_r, xsb_r, ws_r, xsa_l, xsb_l, ws_l,
             xra_r, xrb_r, wr_r, xra_l, xrb_l, wr_l,
             dma_sem):
        my = lax.axis_index("i")
        left = (my + N_DEV - 1) % N_DEV
        right = (my + 1) % N_DEV

        def c(k):
            return (my + 8 * N_DEV + k) % N_DEV

        barrier_sem = pltpu.get_barrier_semaphore()
        for nbr in [left, right]:
            pl.semaphore_signal(
                barrier_sem, inc=1,
                device_id=(nbr,), device_id_type=pl.DeviceIdType.MESH,
            )
        pl.semaphore_wait(barrier_sem, 2)

        pending = []

        def send(src, dst, ssem, rsem, to):
            rdma = pltpu.make_async_remote_copy(
                src_ref=src, dst_ref=dst, send_sem=ssem, recv_sem=rsem,
                device_id=(to,), device_id_type=pl.DeviceIdType.MESH,
            )
            rdma.start()
            pending.append(rdma)
            return rdma

        def accum2(jr, jl, init):
            for r in range(N_DEV):
                d = jnp.dot(
                    xf[jr, 0, pl.ds(r * mc, mc), :], wf[jr, 0],
                    preferred_element_type=jnp.float32,
                ) + jnp.dot(
                    xf[jl, 1, pl.ds(r * mc, mc), :], wf[jl, 1],
                    preferred_element_type=jnp.float32,
                )
                if init:
                    acc[pl.ds(r * mc, mc), :] = d.astype(jnp.bfloat16)
                else:
                    acc[pl.ds(r * mc, mc), :] = (
                        acc[pl.ds(r * mc, mc), :].astype(jnp.float32) + d
                    ).astype(jnp.bfloat16)

        for s in range(N_DEV - 1):
            if s == 0:
                sr_x, sr_w = xa_ref, w_ref.at[pl.ds(0, kh), :]
                sl_x, sl_w = xb_ref, w_ref.at[pl.ds(kh, kh), :]
            else:
                sr_x, sr_w = xf.at[c(-s), 0], wf.at[c(-s), 0]
                sl_x, sl_w = xf.at[c(s), 1], wf.at[c(s), 1]
            step = [
                send(sr_x.at[pl.ds(0, mh), :],
                     xf.at[c(-s), 0, pl.ds(0, mh), :],
                     xsa_r.at[s], xra_r.at[s], right),
                send(sr_w, wf.at[c(-s), 0], ws_r.at[s], wr_r.at[s], right),
                send(sl_x.at[pl.ds(0, mh), :],
                     xf.at[c(s), 1, pl.ds(0, mh), :],
                     xsa_l.at[s], xra_l.at[s], left),
                send(sl_w, wf.at[c(s), 1], ws_l.at[s], wr_l.at[s], left),
                send(sr_x.at[pl.ds(mh, mh), :],
                     xf.at[c(-s), 0, pl.ds(mh, mh), :],
                     xsb_r.at[s], xrb_r.at[s], right),
                send(sl_x.at[pl.ds(mh, mh), :],
                     xf.at[c(s), 1, pl.ds(mh, mh), :],
                     xsb_l.at[s], xrb_l.at[s], left),
            ]
            if s == 0:
                for r in range(N_DEV):
                    acc[pl.ds(r * mc, mc), :] = (
                        jnp.dot(
                            xa_ref[pl.ds(r * mc, mc), :],
                            w_ref[pl.ds(0, kh), :],
                            preferred_element_type=jnp.float32,
                        )
                        + jnp.dot(
                            xb_ref[pl.ds(r * mc, mc), :],
                            w_ref[pl.ds(kh, kh), :],
                            preferred_element_type=jnp.float32,
                        )
                    ).astype(jnp.bfloat16)
            else:
                accum2(c(-s), c(s), False)
            if s < N_DEV - 2:
                for rdma in step:
                    rdma.wait_recv()
            else:
                last_step = step

        scale = sx_ref[0] * sw_ref[0]
        prev_cp = None
        for rdma in last_step[:4]:
            rdma.wait_recv()
        for r in range(N_DEV):
            if r == N_DEV // 2:
                for rdma in last_step[4:]:
                    rdma.wait_recv()
            t = (
                acc[pl.ds(r * mc, mc), :].astype(jnp.float32)
                + jnp.dot(
                    xf[c(1), 0, pl.ds(r * mc, mc), :], wf[c(1), 0],
                    preferred_element_type=jnp.float32,
                )
                + jnp.dot(
                    xf[c(-1), 1, pl.ds(r * mc, mc), :], wf[c(-1), 1],
                    preferred_element_type=jnp.float32,
                )
            )
            y = t * scale
            z = y * (1.0 / (1.0 + jnp.exp(-y)))
            if prev_cp is not None:
                prev_cp.wait()
            tmp[...] = z
            prev_cp = pltpu.make_async_copy(
                tmp, out_ref.at[pl.ds(r * mc, mc), :], dma_sem
            )
            prev_cp.start()
        prev_cp.wait()

        for rdma in pending:
            rdma.wait_send()

    out_shape = jax.ShapeDtypeStruct((m, n), jnp.float32)
    return pl.pallas_call(
        body,
        out_shape=out_shape,
        in_specs=[
            pl.BlockSpec(memory_space=pltpu.VMEM),
            pl.BlockSpec(memory_space=pltpu.VMEM),
            pl.BlockSpec(memory_space=pltpu.VMEM),
            pl.BlockSpec(memory_space=pltpu.SMEM),
            pl.BlockSpec(memory_space=pltpu.SMEM),
        ],
        out_specs=pl.BlockSpec(memory_space=pltpu.MemorySpace.HBM),
        scratch_shapes=[
            pltpu.VMEM((N_DEV, 2, m, kh), jnp.float8_e4m3fn),
            pltpu.VMEM((N_DEV, 2, kh, n), jnp.float8_e4m3fn),
            pltpu.VMEM((m, n), jnp.bfloat16),
            pltpu.VMEM((mc, n), jnp.float32),
            pltpu.SemaphoreType.DMA((N_DEV - 1,)),
            pltpu.SemaphoreType.DMA((N_DEV - 1,)),
            pltpu.SemaphoreType.DMA((N_DEV - 1,)),
            pltpu.SemaphoreType.DMA((N_DEV - 1,)),
            pltpu.SemaphoreType.DMA((N_DEV - 1,)),
            pltpu.SemaphoreType.DMA((N_DEV - 1,)),
            pltpu.SemaphoreType.DMA((N_DEV - 1,)),
            pltpu.SemaphoreType.DMA((N_DEV - 1,)),
            pltpu.SemaphoreType.DMA((N_DEV - 1,)),
            pltpu.SemaphoreType.DMA((N_DEV - 1,)),
            pltpu.SemaphoreType.DMA((N_DEV - 1,)),
            pltpu.SemaphoreType.DMA((N_DEV - 1,)),
            pltpu.SemaphoreType.DMA,
        ],
        compiler_params=pltpu.CompilerParams(
            collective_id=0,
            vmem_limit_bytes=62 * 1024 * 1024,
        ),
    )(x8a, x8b, w8, scale_x, scale_w)


# device time: 168162 ns/iter; 1.1037x vs baseline; 1.0198x over previous
import jax
import jax.numpy as jnp
from jax import lax
from jax.experimental import pallas as pl
from jax.experimental.pallas import tpu as pltpu

N_DEV = 4


def kernel(x, w_mat, scale_x, scale_w):
    m, k_shard = x.shape
    _, n = w_mat.shape
    kh = k_shard // 2
    mc = m // N_DEV

    x8a = x[:, :kh].astype(jnp.float8_e4m3fn)
    x8b = x[:, kh:].astype(jnp.float8_e4m3fn)
    w8 = w_mat.astype(jnp.float8_e4m3fn)

    mh = m // 2

    def body(xa_ref, xb_ref, w_ref, sx_ref, sw_ref, out_ref,
             xf, wf, acc, tmp,
             xsa_r, xsb_r, ws_r, xsa_l, xsb_l, ws_l,
             xra_r, xrb_r, wr_r, xra_l, xrb_l, wr_l,
             dma_sem):
        my = lax.axis_index("i")
        left = (my + N_DEV - 1) % N_DEV
        right = (my + 1) % N_DEV

        def c(k):
            return (my + 8 * N_DEV + k) % N_DEV

        barrier_sem = pltpu.get_barrier_semaphore()
        for nbr in [left, right]:
            pl.semaphore_signal(
                barrier_sem, inc=1,
                device_id=(nbr,), device_id_type=pl.DeviceIdType.MESH,
            )
        pl.semaphore_wait(barrier_sem, 2)

        pending = []

        def send(src, dst, ssem, rsem, to):
            rdma = pltpu.make_async_remote_copy(
                src_ref=src, dst_ref=dst, send_sem=ssem, recv_sem=rsem,
                device_id=(to,), device_id_type=pl.DeviceIdType.MESH,
            )
            rdma.start()
            pending.append(rdma)
            return rdma

        def accum2(jr, jl, init):
            for r in range(N_DEV):
                d = jnp.dot(
                    xf[jr, 0, pl.ds(r * mc, mc), :], wf[jr, 0],
                    preferred_element_type=jnp.float32,
                ) + jnp.dot(
                    xf[jl, 1, pl.ds(r * mc, mc), :], wf[jl, 1],
                    preferred_element_type=jnp.float32,
                )
                if init:
                    acc[pl.ds(r * mc, mc), :] = d.astype(jnp.bfloat16)
                else:
                    acc[pl.ds(r * mc, mc), :] = (
                        acc[pl.ds(r * mc, mc), :].astype(jnp.float32) + d
                    ).astype(jnp.bfloat16)

        def issue_step(s, prev):
            if s == 0:
                sr_x, sr_w = xa_ref, w_ref.at[pl.ds(0, kh), :]
                sl_x, sl_w = xb_ref, w_ref.at[pl.ds(kh, kh), :]
            else:
                sr_x, sr_w = xf.at[c(-s), 0], wf.at[c(-s), 0]
                sl_x, sl_w = xf.at[c(s), 1], wf.at[c(s), 1]
            step = []
            for piece, (src, dst, ssem, rsem, to) in enumerate([
                (sr_x.at[pl.ds(0, mh), :], xf.at[c(-s), 0, pl.ds(0, mh), :],
                 xsa_r.at[s], xra_r.at[s], right),
                (sl_x.at[pl.ds(0, mh), :], xf.at[c(s), 1, pl.ds(0, mh), :],
                 xsa_l.at[s], xra_l.at[s], left),
                (sr_w, wf.at[c(-s), 0], ws_r.at[s], wr_r.at[s], right),
                (sl_w, wf.at[c(s), 1], ws_l.at[s], wr_l.at[s], left),
                (sr_x.at[pl.ds(mh, mh), :], xf.at[c(-s), 0, pl.ds(mh, mh), :],
                 xsb_r.at[s], xrb_r.at[s], right),
                (sl_x.at[pl.ds(mh, mh), :], xf.at[c(s), 1, pl.ds(mh, mh), :],
                 xsb_l.at[s], xrb_l.at[s], left),
            ]):
                if prev is not None:
                    prev[piece].wait_recv()
                step.append(send(src, dst, ssem, rsem, to))
            return step

        prev = None
        for s in range(N_DEV - 1):
            prev = issue_step(s, prev)
            if s == 0:
                for r in range(N_DEV):
                    acc[pl.ds(r * mc, mc), :] = (
                        jnp.dot(
                            xa_ref[pl.ds(r * mc, mc), :],
                            w_ref[pl.ds(0, kh), :],
                            preferred_element_type=jnp.float32,
                        )
                        + jnp.dot(
                            xb_ref[pl.ds(r * mc, mc), :],
                            w_ref[pl.ds(kh, kh), :],
                            preferred_element_type=jnp.float32,
                        )
                    ).astype(jnp.bfloat16)
            else:
                accum2(c(-s), c(s), False)
        last_step = prev

        scale = sx_ref[0] * sw_ref[0]
        prev_cp = None
        for rdma in last_step[:4]:
            rdma.wait_recv()
        for r in range(N_DEV):
            if r == N_DEV // 2:
                for rdma in last_step[4:]:
                    rdma.wait_recv()
            t = (
                acc[pl.ds(r * mc, mc), :].astype(jnp.float32)
                + jnp.dot(
                    xf[c(1), 0, pl.ds(r * mc, mc), :], wf[c(1), 0],
                    preferred_element_type=jnp.float32,
                )
                + jnp.dot(
                    xf[c(-1), 1, pl.ds(r * mc, mc), :], wf[c(-1), 1],
                    preferred_element_type=jnp.float32,
                )
            )
            y = t * scale
            z = y * (1.0 / (1.0 + jnp.exp(-y)))
            if prev_cp is not None:
                prev_cp.wait()
            tmp[...] = z
            prev_cp = pltpu.make_async_copy(
                tmp, out_ref.at[pl.ds(r * mc, mc), :], dma_sem
            )
            prev_cp.start()
        prev_cp.wait()

        for rdma in pending:
            rdma.wait_send()

    out_shape = jax.ShapeDtypeStruct((m, n), jnp.float32)
    return pl.pallas_call(
        body,
        out_shape=out_shape,
        in_specs=[
            pl.BlockSpec(memory_space=pltpu.VMEM),
            pl.BlockSpec(memory_space=pltpu.VMEM),
            pl.BlockSpec(memory_space=pltpu.VMEM),
            pl.BlockSpec(memory_space=pltpu.SMEM),
            pl.BlockSpec(memory_space=pltpu.SMEM),
        ],
        out_specs=pl.BlockSpec(memory_space=pltpu.MemorySpace.HBM),
        scratch_shapes=[
            pltpu.VMEM((N_DEV, 2, m, kh), jnp.float8_e4m3fn),
            pltpu.VMEM((N_DEV, 2, kh, n), jnp.float8_e4m3fn),
            pltpu.VMEM((m, n), jnp.bfloat16),
            pltpu.VMEM((mc, n), jnp.float32),
            pltpu.SemaphoreType.DMA((N_DEV - 1,)),
            pltpu.SemaphoreType.DMA((N_DEV - 1,)),
            pltpu.SemaphoreType.DMA((N_DEV - 1,)),
            pltpu.SemaphoreType.DMA((N_DEV - 1,)),
            pltpu.SemaphoreType.DMA((N_DEV - 1,)),
            pltpu.SemaphoreType.DMA((N_DEV - 1,)),
            pltpu.SemaphoreType.DMA((N_DEV - 1,)),
            pltpu.SemaphoreType.DMA((N_DEV - 1,)),
            pltpu.SemaphoreType.DMA((N_DEV - 1,)),
            pltpu.SemaphoreType.DMA((N_DEV - 1,)),
            pltpu.SemaphoreType.DMA((N_DEV - 1,)),
            pltpu.SemaphoreType.DMA((N_DEV - 1,)),
            pltpu.SemaphoreType.DMA,
        ],
        compiler_params=pltpu.CompilerParams(
            collective_id=0,
            vmem_limit_bytes=62 * 1024 * 1024,
        ),
    )(x8a, x8b, w8, scale_x, scale_w)


# device time: 163247 ns/iter; 1.1370x vs baseline; 1.0301x over previous
import jax
import jax.numpy as jnp
from jax import lax
from jax.experimental import pallas as pl
from jax.experimental.pallas import tpu as pltpu

N_DEV = 4


def kernel(x, w_mat, scale_x, scale_w):
    m, k_shard = x.shape
    _, n = w_mat.shape
    kh = k_shard // 2
    mc = m // N_DEV

    x8a = x[:, :kh].astype(jnp.float8_e4m3fn)
    x8b = x[:, kh:].astype(jnp.float8_e4m3fn)
    w8 = w_mat.astype(jnp.float8_e4m3fn)

    mh = m // 2

    def body(xa_ref, xb_ref, w_ref, sx_ref, sw_ref, out_ref,
             xf, wf, acc, tmp,
             xsa_r, xsb_r, ws_r, xsa_l, xsb_l, ws_l,
             xra_r, xrb_r, wr_r, xra_l, xrb_l, wr_l,
             dma_sems):
        my = lax.axis_index("i")
        left = (my + N_DEV - 1) % N_DEV
        right = (my + 1) % N_DEV

        def c(k):
            return (my + 8 * N_DEV + k) % N_DEV

        barrier_sem = pltpu.get_barrier_semaphore()
        for nbr in [left, right]:
            pl.semaphore_signal(
                barrier_sem, inc=1,
                device_id=(nbr,), device_id_type=pl.DeviceIdType.MESH,
            )
        pl.semaphore_wait(barrier_sem, 2)

        pending = []

        def send(src, dst, ssem, rsem, to):
            rdma = pltpu.make_async_remote_copy(
                src_ref=src, dst_ref=dst, send_sem=ssem, recv_sem=rsem,
                device_id=(to,), device_id_type=pl.DeviceIdType.MESH,
            )
            rdma.start()
            pending.append(rdma)
            return rdma

        def accum2(jr, jl, init):
            for r in range(N_DEV):
                d = jnp.dot(
                    xf[jr, 0, pl.ds(r * mc, mc), :], wf[jr, 0],
                    preferred_element_type=jnp.float32,
                ) + jnp.dot(
                    xf[jl, 1, pl.ds(r * mc, mc), :], wf[jl, 1],
                    preferred_element_type=jnp.float32,
                )
                if init:
                    acc[pl.ds(r * mc, mc), :] = d.astype(jnp.bfloat16)
                else:
                    acc[pl.ds(r * mc, mc), :] = (
                        acc[pl.ds(r * mc, mc), :].astype(jnp.float32) + d
                    ).astype(jnp.bfloat16)

        def issue_step(s, prev):
            if s == 0:
                sr_x, sr_w = xa_ref, w_ref.at[pl.ds(0, kh), :]
                sl_x, sl_w = xb_ref, w_ref.at[pl.ds(kh, kh), :]
            else:
                sr_x, sr_w = xf.at[c(-s), 0], wf.at[c(-s), 0]
                sl_x, sl_w = xf.at[c(s), 1], wf.at[c(s), 1]
            step = []
            for piece, (src, dst, ssem, rsem, to) in enumerate([
                (sr_x.at[pl.ds(0, mh), :], xf.at[c(-s), 0, pl.ds(0, mh), :],
                 xsa_r.at[s], xra_r.at[s], right),
                (sl_x.at[pl.ds(0, mh), :], xf.at[c(s), 1, pl.ds(0, mh), :],
                 xsa_l.at[s], xra_l.at[s], left),
                (sr_w, wf.at[c(-s), 0], ws_r.at[s], wr_r.at[s], right),
                (sl_w, wf.at[c(s), 1], ws_l.at[s], wr_l.at[s], left),
                (sr_x.at[pl.ds(mh, mh), :], xf.at[c(-s), 0, pl.ds(mh, mh), :],
                 xsb_r.at[s], xrb_r.at[s], right),
                (sl_x.at[pl.ds(mh, mh), :], xf.at[c(s), 1, pl.ds(mh, mh), :],
                 xsb_l.at[s], xrb_l.at[s], left),
            ]):
                if prev is not None:
                    prev[piece].wait_recv()
                step.append(send(src, dst, ssem, rsem, to))
            return step

        prev = None
        for s in range(N_DEV - 1):
            prev = issue_step(s, prev)
            if s == 0:
                for r in range(N_DEV):
                    acc[pl.ds(r * mc, mc), :] = (
                        jnp.dot(
                            xa_ref[pl.ds(r * mc, mc), :],
                            w_ref[pl.ds(0, kh), :],
                            preferred_element_type=jnp.float32,
                        )
                        + jnp.dot(
                            xb_ref[pl.ds(r * mc, mc), :],
                            w_ref[pl.ds(kh, kh), :],
                            preferred_element_type=jnp.float32,
                        )
                    ).astype(jnp.bfloat16)
            else:
                accum2(c(-s), c(s), False)
        last_step = prev

        scale = sx_ref[0] * sw_ref[0]
        mt = mc // 2
        cps = [None, None]
        for rdma in last_step[:4]:
            rdma.wait_recv()
        for r in range(2 * N_DEV):
            if r == N_DEV:
                for rdma in last_step[4:]:
                    rdma.wait_recv()
            t = (
                acc[pl.ds(r * mt, mt), :].astype(jnp.float32)
                + jnp.dot(
                    xf[c(1), 0, pl.ds(r * mt, mt), :], wf[c(1), 0],
                    preferred_element_type=jnp.float32,
                )
                + jnp.dot(
                    xf[c(-1), 1, pl.ds(r * mt, mt), :], wf[c(-1), 1],
                    preferred_element_type=jnp.float32,
                )
            )
            y = t * scale
            z = y * (1.0 / (1.0 + jnp.exp(-y)))
            slot = r % 2
            if cps[slot] is not None:
                cps[slot].wait()
            tmp[pl.ds(slot * mt, mt), :] = z
            cps[slot] = pltpu.make_async_copy(
                tmp.at[pl.ds(slot * mt, mt), :],
                out_ref.at[pl.ds(r * mt, mt), :],
                dma_sems.at[slot],
            )
            cps[slot].start()
        cps[0].wait()
        cps[1].wait()

        for rdma in pending:
            rdma.wait_send()

    out_shape = jax.ShapeDtypeStruct((m, n), jnp.float32)
    return pl.pallas_call(
        body,
        out_shape=out_shape,
        in_specs=[
            pl.BlockSpec(memory_space=pltpu.VMEM),
            pl.BlockSpec(memory_space=pltpu.VMEM),
            pl.BlockSpec(memory_space=pltpu.VMEM),
            pl.BlockSpec(memory_space=pltpu.SMEM),
            pl.BlockSpec(memory_space=pltpu.SMEM),
        ],
        out_specs=pl.BlockSpec(memory_space=pltpu.MemorySpace.HBM),
        scratch_shapes=[
            pltpu.VMEM((N_DEV, 2, m, kh), jnp.float8_e4m3fn),
            pltpu.VMEM((N_DEV, 2, kh, n), jnp.float8_e4m3fn),
            pltpu.VMEM((m, n), jnp.bfloat16),
            pltpu.VMEM((mc, n), jnp.float32),
            pltpu.SemaphoreType.DMA((N_DEV - 1,)),
            pltpu.SemaphoreType.DMA((N_DEV - 1,)),
            pltpu.SemaphoreType.DMA((N_DEV - 1,)),
            pltpu.SemaphoreType.DMA((N_DEV - 1,)),
            pltpu.SemaphoreType.DMA((N_DEV - 1,)),
            pltpu.SemaphoreType.DMA((N_DEV - 1,)),
            pltpu.SemaphoreType.DMA((N_DEV - 1,)),
            pltpu.SemaphoreType.DMA((N_DEV - 1,)),
            pltpu.SemaphoreType.DMA((N_DEV - 1,)),
            pltpu.SemaphoreType.DMA((N_DEV - 1,)),
            pltpu.SemaphoreType.DMA((N_DEV - 1,)),
            pltpu.SemaphoreType.DMA((N_DEV - 1,)),
            pltpu.SemaphoreType.DMA((2,)),
        ],
        compiler_params=pltpu.CompilerParams(
            collective_id=0,
            vmem_limit_bytes=62 * 1024 * 1024,
        ),
    )(x8a, x8b, w8, scale_x, scale_w)
